# baseline (speedup 1.0000x reference)
import functools
import math

import jax
import jax.numpy as jnp
from jax import lax
from jax.experimental import pallas as pl
from jax.experimental.pallas import tpu as pltpu
from jax.experimental.pallas import tpu_sc as plsc

import numpy as np

_BIG = np.float32(1e30)
_NEG = np.float32(-3e38)


def _pad_to(x, rows, cols=None, val=0.0):
    pr = rows - x.shape[0]
    if x.ndim == 1:
        return jnp.pad(x, ((0, pr),), constant_values=val)
    pc = (cols - x.shape[1]) if cols is not None else 0
    return jnp.pad(x, ((0, pr), (0, pc)), constant_values=val)


def _fps_body(n, n_samples, xs_s, ys_s, zs_s, x_ref, y_ref, z_ref,
              px_ref, py_ref, pz_ref, dist_ref):
    ri, c = x_ref.shape
    ro = px_ref.shape[0]
    lin = (lax.broadcasted_iota(jnp.int32, (ri, c), 0) * c
           + lax.broadcasted_iota(jnp.int32, (ri, c), 1)).astype(jnp.float32)
    slot = (lax.broadcasted_iota(jnp.int32, (ro, c), 0) * c
            + lax.broadcasted_iota(jnp.int32, (ro, c), 1)).astype(jnp.float32)
    x0 = x_ref[0:1, 0:1]
    y0 = y_ref[0:1, 0:1]
    z0 = z_ref[0:1, 0:1]
    xs = x_ref[...]
    ys = y_ref[...]
    zs = z_ref[...]
    d0 = (xs - x0) ** 2 + (ys - y0) ** 2 + (zs - z0) ** 2
    dist_ref[...] = jnp.where(lin < n, d0, jnp.float32(-1.0))
    px_ref[...] = jnp.where(slot == 0.0, x0, jnp.float32(0.0))
    py_ref[...] = jnp.where(slot == 0.0, y0, jnp.float32(0.0))
    pz_ref[...] = jnp.where(slot == 0.0, z0, jnp.float32(0.0))

    def body(i, carry):
        d = dist_ref[...]
        m = jnp.max(d)
        idx = jnp.min(jnp.where(d == m, lin, jnp.float32(3e7)))
        ii = idx.astype(jnp.int32)
        px = xs_s[ii]
        py = ys_s[ii]
        pz = zs_s[ii]
        nd = (xs - px) ** 2 + (ys - py) ** 2 + (zs - pz) ** 2
        dist_ref[...] = jnp.minimum(d, nd)
        fi = i.astype(jnp.float32)
        px_ref[...] = jnp.where(slot == fi, px, px_ref[...])
        py_ref[...] = jnp.where(slot == fi, py, py_ref[...])
        pz_ref[...] = jnp.where(slot == fi, pz, pz_ref[...])
        return carry

    lax.fori_loop(1, n_samples, body, jnp.int32(0))


def _fps_call(pos, n_samples):
    n = pos.shape[0]
    npad = ((n + 1279) // 1280) * 1280
    opad = ((n_samples + 1279) // 1280) * 1280
    ri, ro = npad // 1280, opad // 1280
    xp = _pad_to(pos[:, 0], npad, val=1e6).reshape(ri, 1280)
    yp = _pad_to(pos[:, 1], npad, val=1e6).reshape(ri, 1280)
    zp = _pad_to(pos[:, 2], npad, val=1e6).reshape(ri, 1280)
    out_sd = [jax.ShapeDtypeStruct((ro, 1280), jnp.float32)] * 3
    sspec = pl.BlockSpec(memory_space=pltpu.SMEM)
    px, py, pz = pl.pallas_call(
        functools.partial(_fps_body, n, n_samples),
        in_specs=[sspec, sspec, sspec,
                  pl.BlockSpec((ri, 1280), lambda: (0, 0)),
                  pl.BlockSpec((ri, 1280), lambda: (0, 0)),
                  pl.BlockSpec((ri, 1280), lambda: (0, 0))],
        out_shape=out_sd,
        scratch_shapes=[pltpu.VMEM((ri, 1280), jnp.float32)],
    )(pos[:, 0], pos[:, 1], pos[:, 2], xp, yp, zp)
    pos_s = jnp.stack([px.reshape(-1)[:n_samples],
                       py.reshape(-1)[:n_samples],
                       pz.reshape(-1)[:n_samples]], axis=1)
    return pos_s


def _select_body(nx, r2, cx_ref, cy_ref, cz_ref, x_ref, y_ref, z_ref,
                 col_ref, val_ref, d2_ref):
    rb = cx_ref.shape[0]
    npx = x_ref.shape[1]
    ci = lax.broadcasted_iota(jnp.int32, (1, npx), 1).astype(jnp.float32)
    tio = lax.broadcasted_iota(jnp.int32, (rb, 64), 1)
    xs = x_ref[...]
    ys = y_ref[...]
    zs = z_ref[...]
    d2 = ((cx_ref[...] - xs) ** 2 + (cy_ref[...] - ys) ** 2
          + (cz_ref[...] - zs) ** 2)
    d2 = jnp.where(ci < nx, d2, _BIG)
    d2_ref[...] = d2
    col_ref[...] = jnp.zeros((rb, 64), jnp.float32)
    val_ref[...] = jnp.zeros((rb, 64), jnp.float32)
    r2f = jnp.float32(r2)

    def cond(carry):
        t, mn = carry
        return jnp.logical_and(t < 64, mn <= r2f)

    def body(carry):
        t, _ = carry
        d = d2_ref[...]
        m = jnp.min(d, axis=1, keepdims=True)
        idx = jnp.min(jnp.where(d == m, ci, jnp.float32(3e7)),
                      axis=1, keepdims=True)
        ok = m <= r2f
        col_ref[...] = jnp.where(tio == t,
                                 jnp.where(ok, idx, jnp.float32(0.0)),
                                 col_ref[...])
        val_ref[...] = jnp.where(tio == t,
                                 jnp.where(ok, jnp.float32(1.0),
                                           jnp.float32(0.0)),
                                 val_ref[...])
        d2_ref[...] = jnp.where(ci == idx, _BIG, d)
        return (t + 1, jnp.min(m))

    lax.while_loop(cond, body, (jnp.int32(0), jnp.float32(-1.0)))


def _select_call(pos_y, pos_x, r):
    ny, nx = pos_y.shape[0], pos_x.shape[0]
    rb = 128
    nyp = ((ny + rb - 1) // rb) * rb
    nxp = ((nx + 1279) // 1280) * 1280
    cx = _pad_to(pos_y[:, 0], nyp, val=1e6).reshape(nyp, 1)
    cy = _pad_to(pos_y[:, 1], nyp, val=1e6).reshape(nyp, 1)
    cz = _pad_to(pos_y[:, 2], nyp, val=1e6).reshape(nyp, 1)
    xr = _pad_to(pos_x[:, 0], nxp, val=1e6).reshape(1, nxp)
    yr = _pad_to(pos_x[:, 1], nxp, val=1e6).reshape(1, nxp)
    zr = _pad_to(pos_x[:, 2], nxp, val=1e6).reshape(1, nxp)
    grid = nyp // rb
    cspec = pl.BlockSpec((rb, 1), lambda i: (i, 0))
    fspec = pl.BlockSpec((1, nxp), lambda i: (0, 0))
    ospec = pl.BlockSpec((rb, 64), lambda i: (i, 0))
    col, valid = pl.pallas_call(
        functools.partial(_select_body, nx, r * r),
        grid=(grid,),
        in_specs=[cspec, cspec, cspec, fspec, fspec, fspec],
        out_specs=[ospec, ospec],
        out_shape=[jax.ShapeDtypeStruct((nyp, 64), jnp.float32)] * 2,
        scratch_shapes=[pltpu.VMEM((rb, nxp), jnp.float32)],
    )(cx, cy, cz, xr, yr, zr)
    return col[:ny].astype(jnp.int32), valid[:ny]


def _sc_gather_call(table, idx, group_chunks):
    e = idx.shape[0]
    d = table.shape[1]
    nw = 32
    bpw = e // nw
    grows = group_chunks * 128
    ngroups = bpw // grows
    mesh = plsc.VectorSubcoreMesh(core_axis_name="c", subcore_axis_name="s")

    @functools.partial(
        pl.kernel,
        mesh=mesh,
        compiler_params=pltpu.CompilerParams(use_tc_tiling_on_sc=False),
        out_type=jax.ShapeDtypeStruct((e, d), jnp.float32),
        scratch_types=[
            pltpu.VMEM((bpw,), jnp.int32),
            pltpu.VMEM((grows, d), jnp.float32),
            pltpu.SemaphoreType.DMA,
        ],
    )
    def k(table_hbm, idx_hbm, out_hbm, idx_v, stage_v, sem):
        wid = lax.axis_index("s") * 2 + lax.axis_index("c")
        base = wid * bpw
        pltpu.sync_copy(idx_hbm.at[pl.ds(base, bpw)], idx_v)
        for g in range(ngroups):
            def fire(j, carry):
                off = g * grows + j * 128
                pltpu.async_copy(
                    table_hbm.at[idx_v.at[pl.ds(off, 128)]],
                    stage_v.at[pl.ds(j * 128, 128), :],
                    sem,
                )
                return carry

            lax.fori_loop(0, group_chunks, fire, jnp.int32(0))
            pltpu.make_async_copy(
                out_hbm.at[pl.ds(0, grows), :], stage_v, sem).wait()
            pltpu.sync_copy(
                stage_v, out_hbm.at[pl.ds(base + g * grows, grows), :])

    return k(table, idx)


def _fold_bn(layers):
    out = []
    for (w, b, g, be, m, v) in layers:
        s = g / jnp.sqrt(v + 1e-5)
        t = be - m * s
        out.append((w, b.reshape(1, -1), s.reshape(1, -1), t.reshape(1, -1)))
    return out


def _mlp3_body(h_ref, py_ref, w1, b1, s1, t1, w2, b2, s2, t2, w3, b3, s3, t3,
               out_ref):
    h = h_ref[...] - py_ref[...]
    for (w, b, s, t) in ((w1, b1, s1, t1), (w2, b2, s2, t2),
                         (w3, b3, s3, t3)):
        h = jnp.dot(h, w[...], preferred_element_type=jnp.float32) + b[...]
        h = jnp.maximum(h, 0.0) * s[...] + t[...]
    out_ref[...] = h


def _mlp3_call(h0, posy, layers):
    e, cin = h0.shape
    eb = posy.shape[0]
    cout = layers[2][0].shape[1]
    grid = e // eb
    specs = [pl.BlockSpec((eb, cin), lambda i: (i, 0)),
             pl.BlockSpec((eb, cin), lambda i: (0, 0))]
    args = [h0, posy]
    w1, b1, s1, t1 = layers[0]
    if w1.shape[0] != cin:
        w1 = jnp.pad(w1, ((0, cin - w1.shape[0]), (0, 0)))
    layers = [(w1, b1, s1, t1)] + list(layers[1:])
    for (w, b, s, t) in layers:
        for a in (w, b, s, t):
            sh = a.shape
            specs.append(pl.BlockSpec(sh, lambda i: (0, 0)))
            args.append(a)
    out = pl.pallas_call(
        _mlp3_body,
        grid=(grid,),
        in_specs=specs,
        out_specs=pl.BlockSpec((eb, cout), lambda i: (i, 0)),
        out_shape=jax.ShapeDtypeStruct((e, cout), jnp.float32),
    )(*args)
    return out


def _conv_body(g_ref, py_ref, v_ref, w1, b1, s1, t1, w2, b2, s2, t2,
               w3, b3, s3, t3, out_ref):
    rb, cout = out_ref.shape
    py = py_ref[...]

    def jloop(j, acc):
        h = g_ref[j] - py
        for (w, b, s, t) in ((w1, b1, s1, t1), (w2, b2, s2, t2),
                             (w3, b3, s3, t3)):
            h = jnp.dot(h, w[...], preferred_element_type=jnp.float32) + b[...]
            h = jnp.maximum(h, 0.0) * s[...] + t[...]
        vj = v_ref[j]
        return jnp.maximum(acc, jnp.where(vj > 0.5, h, _NEG))

    out_ref[...] = lax.fori_loop(0, 64, jloop,
                                 jnp.full((rb, cout), _NEG, jnp.float32))


def _conv_call(g3, posy, v3, layers):
    k, nyp, cin = g3.shape
    cout = layers[2][0].shape[1]
    rb = 128
    grid = nyp // rb
    w1, b1, s1, t1 = layers[0]
    if w1.shape[0] != cin:
        w1 = jnp.pad(w1, ((0, cin - w1.shape[0]), (0, 0)))
    layers = [(w1, b1, s1, t1)] + list(layers[1:])
    specs = [pl.BlockSpec((k, rb, cin), lambda i: (0, i, 0)),
             pl.BlockSpec((rb, cin), lambda i: (i, 0)),
             pl.BlockSpec((k, rb, 1), lambda i: (0, i, 0))]
    args = [g3, posy, v3]
    for (w, b, s, t) in layers:
        for a in (w, b, s, t):
            sh = a.shape
            specs.append(pl.BlockSpec(sh, lambda i: (0, 0)))
            args.append(a)
    out = pl.pallas_call(
        _conv_body,
        grid=(grid,),
        in_specs=specs,
        out_specs=pl.BlockSpec((rb, cout), lambda i: (i, 0)),
        out_shape=jax.ShapeDtypeStruct((nyp, cout), jnp.float32),
    )(*args)
    return out


def _maxpool_body(h_ref, v_ref, out_ref):
    rb, c = out_ref.shape

    def body(j, acc):
        hj = h_ref[j]
        vj = v_ref[j]
        return jnp.maximum(acc, jnp.where(vj > 0.5, hj, _NEG))

    out_ref[...] = lax.fori_loop(0, 64, body, jnp.full((rb, c), _NEG))


def _maxpool_call(h3d, valid3d):
    k, nyp, c = h3d.shape
    rb = 128
    grid = nyp // rb
    out = pl.pallas_call(
        _maxpool_body,
        grid=(grid,),
        in_specs=[pl.BlockSpec((k, rb, c), lambda i: (0, i, 0)),
                  pl.BlockSpec((k, rb, 1), lambda i: (0, i, 0))],
        out_specs=pl.BlockSpec((rb, c), lambda i: (i, 0)),
        out_shape=jax.ShapeDtypeStruct((nyp, c), jnp.float32),
    )(h3d, valid3d)
    return out


def _glob_body(nrow, hin_ref, w1, b1, s1, t1, w2, b2, s2, t2, w3, b3, s3, t3,
               fw1, fb1, fw2, fb2, fw3, fb3, out_ref):
    h = hin_ref[...]
    for (w, b, s, t) in ((w1, b1, s1, t1), (w2, b2, s2, t2),
                         (w3, b3, s3, t3)):
        h = jnp.dot(h, w[...], preferred_element_type=jnp.float32) + b[...]
        h = jnp.maximum(h, 0.0) * s[...] + t[...]
    rows = lax.broadcasted_iota(jnp.int32, h.shape, 0)
    h = jnp.where(rows < nrow, h, _NEG)
    g = jnp.max(h, axis=0, keepdims=True)
    z = jnp.maximum(jnp.dot(g, fw1[...], preferred_element_type=jnp.float32)
                    + fb1[...], 0.0)
    z = jnp.maximum(jnp.dot(z, fw2[...], preferred_element_type=jnp.float32)
                    + fb2[...], 0.0)
    o = jnp.dot(z, fw3[...], preferred_element_type=jnp.float32) + fb3[...]
    out_ref[...] = jax.nn.sigmoid(o)


def _glob_call(hin, glayers, fc1, fc2, fc3):
    nrow = hin.shape[0]
    nyp = ((nrow + 127) // 128) * 128
    cin = hin.shape[1]
    cinp = ((cin + 127) // 128) * 128
    hp = _pad_to(hin, nyp, cinp)
    args = [hp]
    for (w, b, s, t) in glayers:
        wp = jnp.pad(w, ((0, cinp - w.shape[0]), (0, 0))) if w.shape[0] != cinp \
            else w
        args.extend([wp, b, s, t])
        cinp = w.shape[1]
    w1, b1 = fc1
    w2, b2 = fc2
    w3, b3 = fc3
    w3p = jnp.pad(w3, ((0, 0), (0, 128 - w3.shape[1])))
    b3p = jnp.pad(b3.reshape(1, -1), ((0, 0), (0, 128 - b3.shape[0])))
    args.extend([w1, b1.reshape(1, -1), w2, b2.reshape(1, -1), w3p, b3p])
    specs = [pl.BlockSpec(a.shape, lambda i: (0, 0)) for a in args]
    out = pl.pallas_call(
        functools.partial(_glob_body, nrow),
        grid=(1,),
        in_specs=specs,
        out_specs=pl.BlockSpec((1, 128), lambda i: (0, 0)),
        out_shape=jax.ShapeDtypeStruct((1, 128), jnp.float32),
    )(*args)
    return out[0, 0]


def _point_conv(x_feat, pos_x, pos_y, col, valid, layers, cin_pad,
                group_chunks):
    ny, k = col.shape
    nyp = ((ny + 127) // 128) * 128
    if x_feat.ndim == 1:
        x_feat = x_feat[:, None]
    cf = x_feat.shape[1]
    table = jnp.concatenate([x_feat, pos_x], axis=1)
    table = jnp.pad(table, ((0, 0), (0, cin_pad - table.shape[1])))
    colt = jnp.pad(col.T, ((0, 0), (0, nyp - ny))).reshape(-1)
    g = _sc_gather_call(table, colt, group_chunks)
    posy = jnp.pad(pos_y, ((0, nyp - ny), (0, 0)))
    posy = jnp.pad(jnp.concatenate(
        [jnp.zeros((nyp, cf), jnp.float32), posy], axis=1),
        ((0, 0), (0, cin_pad - cf - 3)))
    v3 = jnp.pad(valid.T[:, :, None], ((0, 0), (0, nyp - ny), (0, 0)))
    out = _conv_call(g.reshape(k, nyp, cin_pad), posy, v3, layers)
    return out[:ny]


def kernel(points, features, params):
    n = points.shape[0]
    n1 = (n + 1) // 2
    pos1 = _fps_call(points, n1)
    col1, valid1 = _select_call(pos1, points, 0.03)
    sa1 = _fold_bn(params["sa1"])
    x1 = _point_conv(features, points, pos1, col1, valid1, sa1, 16, 40)

    n2 = (n1 + 3) // 4
    pos2 = _fps_call(pos1, n2)
    col2, valid2 = _select_call(pos2, pos1, 0.2)
    sa2 = _fold_bn(params["sa2"])
    x2 = _point_conv(x1, pos1, pos2, col2, valid2, sa2, 144, 5)

    hin = jnp.concatenate([x2, pos2], axis=1)
    glob = _fold_bn(params["glob"])
    out = _glob_call(hin, glob, params["fc1"], params["fc2"], params["fc3"])
    return out


# speedup vs baseline: 1.1535x; 1.1535x over previous
import functools
import math

import jax
import jax.numpy as jnp
from jax import lax
from jax.experimental import pallas as pl
from jax.experimental.pallas import tpu as pltpu
from jax.experimental.pallas import tpu_sc as plsc

import numpy as np

_BIG = np.float32(1e30)
_NEG = np.float32(-3e38)


def _pad_to(x, rows, cols=None, val=0.0):
    pr = rows - x.shape[0]
    if x.ndim == 1:
        return jnp.pad(x, ((0, pr),), constant_values=val)
    pc = (cols - x.shape[1]) if cols is not None else 0
    return jnp.pad(x, ((0, pr), (0, pc)), constant_values=val)


def _fps_body(n, n_samples, xs_s, ys_s, zs_s, x_ref, y_ref, z_ref,
              px_ref, py_ref, pz_ref, dist_ref):
    ri, c = x_ref.shape
    ro = px_ref.shape[0]
    lin = (lax.broadcasted_iota(jnp.int32, (ri, c), 0) * c
           + lax.broadcasted_iota(jnp.int32, (ri, c), 1)).astype(jnp.float32)
    slot = (lax.broadcasted_iota(jnp.int32, (ro, c), 0) * c
            + lax.broadcasted_iota(jnp.int32, (ro, c), 1)).astype(jnp.float32)
    x0 = x_ref[0:1, 0:1]
    y0 = y_ref[0:1, 0:1]
    z0 = z_ref[0:1, 0:1]
    xs = x_ref[...]
    ys = y_ref[...]
    zs = z_ref[...]
    d0 = (xs - x0) ** 2 + (ys - y0) ** 2 + (zs - z0) ** 2
    dist_ref[...] = jnp.where(lin < n, d0, jnp.float32(-1.0))
    px_ref[...] = jnp.where(slot == 0.0, x0, jnp.float32(0.0))
    py_ref[...] = jnp.where(slot == 0.0, y0, jnp.float32(0.0))
    pz_ref[...] = jnp.where(slot == 0.0, z0, jnp.float32(0.0))

    def body(i, carry):
        d = dist_ref[...]
        m = jnp.max(d)
        idx = jnp.min(jnp.where(d == m, lin, jnp.float32(3e7)))
        ii = idx.astype(jnp.int32)
        px = xs_s[ii]
        py = ys_s[ii]
        pz = zs_s[ii]
        nd = (xs - px) ** 2 + (ys - py) ** 2 + (zs - pz) ** 2
        dist_ref[...] = jnp.minimum(d, nd)
        fi = i.astype(jnp.float32)
        px_ref[...] = jnp.where(slot == fi, px, px_ref[...])
        py_ref[...] = jnp.where(slot == fi, py, py_ref[...])
        pz_ref[...] = jnp.where(slot == fi, pz, pz_ref[...])
        return carry

    lax.fori_loop(1, n_samples, body, jnp.int32(0))


def _fps_call(pos, n_samples):
    n = pos.shape[0]
    npad = ((n + 1279) // 1280) * 1280
    opad = ((n_samples + 1279) // 1280) * 1280
    ri, ro = npad // 1280, opad // 1280
    xp = _pad_to(pos[:, 0], npad, val=1e6).reshape(ri, 1280)
    yp = _pad_to(pos[:, 1], npad, val=1e6).reshape(ri, 1280)
    zp = _pad_to(pos[:, 2], npad, val=1e6).reshape(ri, 1280)
    out_sd = [jax.ShapeDtypeStruct((ro, 1280), jnp.float32)] * 3
    sspec = pl.BlockSpec(memory_space=pltpu.SMEM)
    px, py, pz = pl.pallas_call(
        functools.partial(_fps_body, n, n_samples),
        in_specs=[sspec, sspec, sspec,
                  pl.BlockSpec((ri, 1280), lambda: (0, 0)),
                  pl.BlockSpec((ri, 1280), lambda: (0, 0)),
                  pl.BlockSpec((ri, 1280), lambda: (0, 0))],
        out_shape=out_sd,
        scratch_shapes=[pltpu.VMEM((ri, 1280), jnp.float32)],
    )(pos[:, 0], pos[:, 1], pos[:, 2], xp, yp, zp)
    pos_s = jnp.stack([px.reshape(-1)[:n_samples],
                       py.reshape(-1)[:n_samples],
                       pz.reshape(-1)[:n_samples]], axis=1)
    return pos_s


def _select_body(nx, r2, cx_ref, cy_ref, cz_ref, x_ref, y_ref, z_ref,
                 col_ref, val_ref, d2_ref):
    rb = cx_ref.shape[0]
    npx = x_ref.shape[1]
    ci = lax.broadcasted_iota(jnp.int32, (1, npx), 1).astype(jnp.float32)
    tio = lax.broadcasted_iota(jnp.int32, (rb, 64), 1)
    xs = x_ref[...]
    ys = y_ref[...]
    zs = z_ref[...]
    d2 = ((cx_ref[...] - xs) ** 2 + (cy_ref[...] - ys) ** 2
          + (cz_ref[...] - zs) ** 2)
    d2 = jnp.where(ci < nx, d2, _BIG)
    d2_ref[...] = d2
    col_ref[...] = jnp.zeros((rb, 64), jnp.float32)
    val_ref[...] = jnp.zeros((rb, 64), jnp.float32)
    r2f = jnp.float32(r2)

    def cond(carry):
        t, mn = carry
        return jnp.logical_and(t < 64, mn <= r2f)

    def body(carry):
        t, _ = carry
        d = d2_ref[...]
        m = jnp.min(d, axis=1, keepdims=True)
        idx = jnp.min(jnp.where(d == m, ci, jnp.float32(3e7)),
                      axis=1, keepdims=True)
        ok = m <= r2f
        col_ref[...] = jnp.where(tio == t,
                                 jnp.where(ok, idx, jnp.float32(0.0)),
                                 col_ref[...])
        val_ref[...] = jnp.where(tio == t,
                                 jnp.where(ok, jnp.float32(1.0),
                                           jnp.float32(0.0)),
                                 val_ref[...])
        d2_ref[...] = jnp.where(ci == idx, _BIG, d)
        return (t + 1, jnp.min(m))

    lax.while_loop(cond, body, (jnp.int32(0), jnp.float32(-1.0)))


def _select_call(pos_y, pos_x, r):
    ny, nx = pos_y.shape[0], pos_x.shape[0]
    rb = 128
    nyp = ((ny + rb - 1) // rb) * rb
    nxp = ((nx + 1279) // 1280) * 1280
    cx = _pad_to(pos_y[:, 0], nyp, val=1e6).reshape(nyp, 1)
    cy = _pad_to(pos_y[:, 1], nyp, val=1e6).reshape(nyp, 1)
    cz = _pad_to(pos_y[:, 2], nyp, val=1e6).reshape(nyp, 1)
    xr = _pad_to(pos_x[:, 0], nxp, val=1e6).reshape(1, nxp)
    yr = _pad_to(pos_x[:, 1], nxp, val=1e6).reshape(1, nxp)
    zr = _pad_to(pos_x[:, 2], nxp, val=1e6).reshape(1, nxp)
    grid = nyp // rb
    cspec = pl.BlockSpec((rb, 1), lambda i: (i, 0))
    fspec = pl.BlockSpec((1, nxp), lambda i: (0, 0))
    ospec = pl.BlockSpec((rb, 64), lambda i: (i, 0))
    col, valid = pl.pallas_call(
        functools.partial(_select_body, nx, r * r),
        grid=(grid,),
        in_specs=[cspec, cspec, cspec, fspec, fspec, fspec],
        out_specs=[ospec, ospec],
        out_shape=[jax.ShapeDtypeStruct((nyp, 64), jnp.float32)] * 2,
        scratch_shapes=[pltpu.VMEM((rb, nxp), jnp.float32)],
    )(cx, cy, cz, xr, yr, zr)
    return col[:ny].astype(jnp.int32), valid[:ny]


def _sc_gather_call(table, idx, group_chunks):
    e = idx.shape[0]
    d = table.shape[1]
    nw = 32
    bpw = e // nw
    grows = group_chunks * 128
    ngroups = bpw // grows
    mesh = plsc.VectorSubcoreMesh(core_axis_name="c", subcore_axis_name="s")

    @functools.partial(
        pl.kernel,
        mesh=mesh,
        compiler_params=pltpu.CompilerParams(use_tc_tiling_on_sc=False),
        out_type=jax.ShapeDtypeStruct((e, d), jnp.float32),
        scratch_types=[
            pltpu.VMEM((bpw,), jnp.int32),
            pltpu.VMEM((grows, d), jnp.float32),
            pltpu.SemaphoreType.DMA,
        ],
    )
    def k(table_hbm, idx_hbm, out_hbm, idx_v, stage_v, sem):
        wid = lax.axis_index("s") * 2 + lax.axis_index("c")
        base = wid * bpw
        pltpu.sync_copy(idx_hbm.at[pl.ds(base, bpw)], idx_v)
        for g in range(ngroups):
            def fire(j, carry):
                off = g * grows + j * 128
                pltpu.async_copy(
                    table_hbm.at[idx_v.at[pl.ds(off, 128)]],
                    stage_v.at[pl.ds(j * 128, 128), :],
                    sem,
                )
                return carry

            lax.fori_loop(0, group_chunks, fire, jnp.int32(0))
            pltpu.make_async_copy(
                out_hbm.at[pl.ds(0, grows), :], stage_v, sem).wait()
            pltpu.sync_copy(
                stage_v, out_hbm.at[pl.ds(base + g * grows, grows), :])

    return k(table, idx)


def _fold_bn(layers):
    out = []
    for (w, b, g, be, m, v) in layers:
        s = g / jnp.sqrt(v + 1e-5)
        t = be - m * s
        out.append((w, b.reshape(1, -1), s.reshape(1, -1), t.reshape(1, -1)))
    return out


def _mlp3_body(h_ref, py_ref, w1, b1, s1, t1, w2, b2, s2, t2, w3, b3, s3, t3,
               out_ref):
    h = h_ref[...] - py_ref[...]
    for (w, b, s, t) in ((w1, b1, s1, t1), (w2, b2, s2, t2),
                         (w3, b3, s3, t3)):
        h = jnp.dot(h, w[...], preferred_element_type=jnp.float32) + b[...]
        h = jnp.maximum(h, 0.0) * s[...] + t[...]
    out_ref[...] = h


def _mlp3_call(h0, posy, layers):
    e, cin = h0.shape
    eb = posy.shape[0]
    cout = layers[2][0].shape[1]
    grid = e // eb
    specs = [pl.BlockSpec((eb, cin), lambda i: (i, 0)),
             pl.BlockSpec((eb, cin), lambda i: (0, 0))]
    args = [h0, posy]
    w1, b1, s1, t1 = layers[0]
    if w1.shape[0] != cin:
        w1 = jnp.pad(w1, ((0, cin - w1.shape[0]), (0, 0)))
    layers = [(w1, b1, s1, t1)] + list(layers[1:])
    for (w, b, s, t) in layers:
        for a in (w, b, s, t):
            sh = a.shape
            specs.append(pl.BlockSpec(sh, lambda i: (0, 0)))
            args.append(a)
    out = pl.pallas_call(
        _mlp3_body,
        grid=(grid,),
        in_specs=specs,
        out_specs=pl.BlockSpec((eb, cout), lambda i: (i, 0)),
        out_shape=jax.ShapeDtypeStruct((e, cout), jnp.float32),
    )(*args)
    return out


def _maxpool_body(h_ref, v_ref, out_ref):
    rb, c = out_ref.shape

    def body(j, acc):
        hj = h_ref[j]
        vj = v_ref[j]
        return jnp.maximum(acc, jnp.where(vj > 0.5, hj, _NEG))

    out_ref[...] = lax.fori_loop(0, 64, body, jnp.full((rb, c), _NEG))


def _maxpool_call(h3d, valid3d):
    k, nyp, c = h3d.shape
    rb = 128
    grid = nyp // rb
    out = pl.pallas_call(
        _maxpool_body,
        grid=(grid,),
        in_specs=[pl.BlockSpec((k, rb, c), lambda i: (0, i, 0)),
                  pl.BlockSpec((k, rb, 1), lambda i: (0, i, 0))],
        out_specs=pl.BlockSpec((rb, c), lambda i: (i, 0)),
        out_shape=jax.ShapeDtypeStruct((nyp, c), jnp.float32),
    )(h3d, valid3d)
    return out


def _glob_body(nrow, hin_ref, w1, b1, s1, t1, w2, b2, s2, t2, w3, b3, s3, t3,
               fw1, fb1, fw2, fb2, fw3, fb3, out_ref):
    h = hin_ref[...]
    for (w, b, s, t) in ((w1, b1, s1, t1), (w2, b2, s2, t2),
                         (w3, b3, s3, t3)):
        h = jnp.dot(h, w[...], preferred_element_type=jnp.float32) + b[...]
        h = jnp.maximum(h, 0.0) * s[...] + t[...]
    rows = lax.broadcasted_iota(jnp.int32, h.shape, 0)
    h = jnp.where(rows < nrow, h, _NEG)
    g = jnp.max(h, axis=0, keepdims=True)
    z = jnp.maximum(jnp.dot(g, fw1[...], preferred_element_type=jnp.float32)
                    + fb1[...], 0.0)
    z = jnp.maximum(jnp.dot(z, fw2[...], preferred_element_type=jnp.float32)
                    + fb2[...], 0.0)
    o = jnp.dot(z, fw3[...], preferred_element_type=jnp.float32) + fb3[...]
    out_ref[...] = jax.nn.sigmoid(o)


def _glob_call(hin, glayers, fc1, fc2, fc3):
    nrow = hin.shape[0]
    nyp = ((nrow + 127) // 128) * 128
    cin = hin.shape[1]
    cinp = ((cin + 127) // 128) * 128
    hp = _pad_to(hin, nyp, cinp)
    args = [hp]
    for (w, b, s, t) in glayers:
        wp = jnp.pad(w, ((0, cinp - w.shape[0]), (0, 0))) if w.shape[0] != cinp \
            else w
        args.extend([wp, b, s, t])
        cinp = w.shape[1]
    w1, b1 = fc1
    w2, b2 = fc2
    w3, b3 = fc3
    w3p = jnp.pad(w3, ((0, 0), (0, 128 - w3.shape[1])))
    b3p = jnp.pad(b3.reshape(1, -1), ((0, 0), (0, 128 - b3.shape[0])))
    args.extend([w1, b1.reshape(1, -1), w2, b2.reshape(1, -1), w3p, b3p])
    specs = [pl.BlockSpec(a.shape, lambda i: (0, 0)) for a in args]
    out = pl.pallas_call(
        functools.partial(_glob_body, nrow),
        grid=(1,),
        in_specs=specs,
        out_specs=pl.BlockSpec((1, 128), lambda i: (0, 0)),
        out_shape=jax.ShapeDtypeStruct((1, 128), jnp.float32),
    )(*args)
    return out[0, 0]


def _point_conv(x_feat, pos_x, pos_y, col, valid, layers, cin_pad,
                group_chunks):
    ny, k = col.shape
    nyp = ((ny + 127) // 128) * 128
    if x_feat.ndim == 1:
        x_feat = x_feat[:, None]
    cf = x_feat.shape[1]
    table = jnp.concatenate([x_feat, pos_x], axis=1)
    table = jnp.pad(table, ((0, 0), (0, cin_pad - table.shape[1])))
    colt = jnp.pad(col.T, ((0, 0), (0, nyp - ny))).reshape(-1)
    g = _sc_gather_call(table, colt, group_chunks)
    posy = jnp.pad(pos_y, ((0, nyp - ny), (0, 0)))
    posy = jnp.pad(jnp.concatenate(
        [jnp.zeros((nyp, cf), jnp.float32), posy], axis=1),
        ((0, 0), (0, cin_pad - cf - 3)))
    h3 = _mlp3_call(g, posy, layers)
    cout = h3.shape[1]
    v3 = jnp.pad(valid.T[:, :, None], ((0, 0), (0, nyp - ny), (0, 0)))
    out = _maxpool_call(h3.reshape(k, nyp, cout), v3)
    return out[:ny]


def kernel(points, features, params):
    n = points.shape[0]
    n1 = (n + 1) // 2
    pos1 = _fps_call(points, n1)
    col1, valid1 = _select_call(pos1, points, 0.03)
    sa1 = _fold_bn(params["sa1"])
    x1 = _point_conv(features, points, pos1, col1, valid1, sa1, 16, 40)

    n2 = (n1 + 3) // 4
    pos2 = _fps_call(pos1, n2)
    col2, valid2 = _select_call(pos2, pos1, 0.2)
    sa2 = _fold_bn(params["sa2"])
    x2 = _point_conv(x1, pos1, pos2, col2, valid2, sa2, 144, 5)

    hin = jnp.concatenate([x2, pos2], axis=1)
    glob = _fold_bn(params["glob"])
    out = _glob_call(hin, glob, params["fc1"], params["fc2"], params["fc3"])
    return out
